# baseline (speedup 1.0000x reference)
import functools

import jax
import jax.numpy as jnp
from jax import lax
from jax.experimental import pallas as pl
from jax.experimental.pallas import tpu as pltpu
from jax.experimental.pallas import tpu_sc as plsc

_K_LIST = [16, 128, 16, 256]
_G = 128
_QPAD = 10240
_NW = 32
_NQW = _QPAD // _NW
_ROWBATCH = 16


def _elu(x):
    return jax.nn.elu(x)


def _bn(x, g, b, eps=1e-5):
    m = jnp.mean(x, axis=0)
    v = jnp.var(x, axis=0)
    return (x - m) / jnp.sqrt(v + eps) * g + b


def _thresh_body(k, st_ref, s_ref, t_ref, dt_ref):
    stb = st_ref[...]
    d0 = s_ref[:, 0:1] - stb[0, 0:1, :]
    d1 = s_ref[:, 1:2] - stb[0, 1:2, :]
    d2 = s_ref[:, 2:3] - stb[0, 2:3, :]
    dt = d0 * d0 + d1 * d1 + d2 * d2
    dt_ref[...] = dt
    di0 = lax.bitcast_convert_type(dt, jnp.int32)
    hi0 = jnp.max(di0, axis=0, keepdims=True)
    lo0 = jnp.zeros_like(hi0)

    def bis(_, c):
        lo, hi = c
        mid = lo + ((hi - lo) >> 1)
        di = lax.bitcast_convert_type(dt_ref[...], jnp.int32)
        cnt = jnp.sum(jnp.where(di <= mid, 1.0, 0.0), axis=0, keepdims=True)
        ge = cnt >= float(k)
        return jnp.where(ge, lo, mid + 1), jnp.where(ge, mid, hi)

    _, tb = lax.fori_loop(0, 31, bis, (lo0, hi0))
    t_ref[...] = lax.bitcast_convert_type(tb, jnp.float32).reshape(1, 1, _G)


@functools.partial(jax.jit, static_argnames=('k',))
def _thresholds(stg, s, k):
    n = s.shape[0]
    ngrid = _QPAD // _G
    t3 = pl.pallas_call(
        functools.partial(_thresh_body, k),
        grid=(ngrid,),
        in_specs=[
            pl.BlockSpec((1, 3, _G), lambda g: (g, 0, 0)),
            pl.BlockSpec((n, 3), lambda g: (0, 0)),
        ],
        out_specs=pl.BlockSpec((1, 1, _G), lambda g: (g, 0, 0)),
        out_shape=jax.ShapeDtypeStruct((ngrid, 1, _G), jnp.float32),
        scratch_shapes=[pltpu.VMEM((n, _G), jnp.float32)],
    )(stg, s)
    return t3.reshape(_QPAD)


def _lane_splat(vec, i):
    idx = jnp.full((16, 1), i, jnp.int32)
    return lax.gather(
        vec, idx,
        dimension_numbers=lax.GatherDimensionNumbers(
            offset_dims=(), collapsed_slice_dims=(0,), start_index_map=(0,)),
        slice_sizes=(1,), mode=lax.GatherScatterMode.PROMISE_IN_BOUNDS)


def _sc_body(C, n_real, s0h, s1h, s2h, th, d2h, jh,
             s0, s1, s2, tv, rowd, rowj):
    NC = 2
    wid = lax.axis_index("s") * NC + lax.axis_index("c")
    pltpu.sync_copy(s0h, s0)
    pltpu.sync_copy(s1h, s1)
    pltpu.sync_copy(s2h, s2)
    pltpu.sync_copy(th, tv)
    qbase = wid * _NQW
    lane = lax.iota(jnp.int32, 16)
    nchunk = n_real // 16
    sent_d = jnp.full((16,), 1e30, jnp.float32)
    sent_j = jnp.zeros((16,), jnp.int32)
    nbuf = (_ROWBATCH * C) // 16

    def qgroup(gi, _):
        def clr(bi, _):
            rowd[pl.ds(bi * 16, 16)] = sent_d
            rowj[pl.ds(bi * 16, 16)] = sent_j
            return 0
        lax.fori_loop(0, nbuf, clr, 0)

        qg = qbase + gi * _ROWBATCH
        qv0 = s0[pl.ds(qg, 16)]
        qv1 = s1[pl.ds(qg, 16)]
        qv2 = s2[pl.ds(qg, 16)]
        tvv = tv[pl.ds(qg, 16)]
        for qi in range(_ROWBATCH):
            q0 = _lane_splat(qv0, qi)
            q1 = _lane_splat(qv1, qi)
            q2 = _lane_splat(qv2, qi)
            tq = _lane_splat(tvv, qi)
            rbase = qi * C
            cap = jnp.full((16,), qi * C + C - 1, jnp.int32)

            def chunk(c, obase, q0=q0, q1=q1, q2=q2, tq=tq,
                      rbase=rbase, cap=cap):
                e0 = s0[pl.ds(c * 16, 16)] - q0
                e1 = s1[pl.ds(c * 16, 16)] - q1
                e2 = s2[pl.ds(c * 16, 16)] - q2
                dv = e0 * e0 + e1 * e1 + e2 * e2
                m = dv <= tq
                mi = jnp.where(m, 1, 0)
                cum = plsc.cumsum(mi)
                pos = jnp.minimum(obase + (cum - mi), cap)
                plsc.store_scatter(rowd, [pos], dv, mask=m)
                plsc.store_scatter(rowj, [pos], lane + c * 16, mask=m)
                pc = plsc.all_reduce_population_count(m)
                return jnp.minimum(obase + pc, cap)

            lax.fori_loop(0, nchunk, chunk,
                          jnp.full((16,), rbase, jnp.int32))

        rq = qbase + gi * _ROWBATCH
        pltpu.sync_copy(rowd, d2h.at[pl.ds(rq * C, _ROWBATCH * C)])
        pltpu.sync_copy(rowj, jh.at[pl.ds(rq * C, _ROWBATCH * C)])
        return 0

    lax.fori_loop(0, _NQW // _ROWBATCH, qgroup, 0)


@functools.partial(jax.jit, static_argnames=('C', 'n_real'))
def _sc_compact(s0, s1, s2, t, C, n_real):
    mesh = plsc.VectorSubcoreMesh(core_axis_name="c", subcore_axis_name="s")
    kfn = pl.kernel(
        functools.partial(_sc_body, C, n_real),
        out_type=[
            jax.ShapeDtypeStruct((_QPAD * C,), jnp.float32),
            jax.ShapeDtypeStruct((_QPAD * C,), jnp.int32),
        ],
        mesh=mesh,
        compiler_params=pltpu.CompilerParams(needs_layout_passes=False),
        scratch_types=[
            pltpu.VMEM((_QPAD,), jnp.float32),
            pltpu.VMEM((_QPAD,), jnp.float32),
            pltpu.VMEM((_QPAD,), jnp.float32),
            pltpu.VMEM((_QPAD,), jnp.float32),
            pltpu.VMEM((_ROWBATCH * C,), jnp.float32),
            pltpu.VMEM((_ROWBATCH * C,), jnp.int32),
        ],
    )
    return kfn(s0, s1, s2, t)


def _knn_pallas(s, k):
    n = s.shape[0]
    C = k + 16
    stp = jnp.zeros((3, _QPAD), s.dtype).at[:, :n].set(s.T)
    stg = stp.reshape(3, _QPAD // _G, _G).transpose(1, 0, 2)
    t = _thresholds(stg, s, k)
    d2f, jf = _sc_compact(stp[0], stp[1], stp[2], t, C, n)
    d2c = d2f.reshape(_QPAD, C)[:n]
    jc = jf.reshape(_QPAD, C)[:n]
    nd2, pos = lax.top_k(-d2c, k)
    idx = jnp.take_along_axis(jc, pos, axis=1)
    return -nd2, idx


def _gravnet_conv(x, p, k):
    s = x @ p['Ws'] + p['bs']
    h = x @ p['Wh'] + p['bh']
    d2, idx = _knn_pallas(s, k)
    w = jnp.exp(-10.0 * d2)[..., None]
    n = h.shape[0]
    hg = jnp.take(h, idx.reshape(-1), axis=0).reshape(n, k, h.shape[1])
    msg = hg * w
    mean_agg = jnp.mean(msg, axis=1)
    max_agg = jnp.max(msg, axis=1)
    out = jnp.concatenate([x, mean_agg, max_agg], axis=1) @ p['Wo'] + p['bo']
    return out, s


def _block_fn(x_in, p, k):
    x = _elu(x_in @ p['pre1W'] + p['pre1b'])
    x = _elu(x @ p['pre2W'] + p['pre2b'])
    x = _bn(x, p['bn1g'], p['bn1b'])
    x_input = x
    xgn, s = _gravnet_conv(x, p, k)
    x = jnp.concatenate([xgn, s, x_input], axis=1)
    x = _elu(x @ p['post1W'] + p['post1b'])
    x = _elu(x @ p['post2W'] + p['post2b'])
    x = _bn(x, p['bn2g'], p['bn2b'])
    return x


def kernel(x, step_count, params):
    x = _bn(x, params['bn0_g'], params['bn0_b'])
    x = x @ params['Wd1']
    allfeat = [x]
    cur = x
    for i, k in enumerate(_K_LIST):
        xi = _block_fn(cur, params['block%d' % i], k)
        allfeat.append(xi)
        cur = jnp.concatenate(allfeat, axis=1)
    x = jnp.concatenate(allfeat, axis=-1)
    for j in range(3):
        x = _elu(x @ params['denseW%d' % j] + params['denseb%d' % j])
    x = _bn(x, params['bn2_g'], params['bn2_b'])
    x_cluster = x @ params['Wclust']
    pred_energy = 1.0 + _elu(x @ params['Wpe'])
    beta = x @ params['Wbeta'] + params['bbeta']
    out = jnp.concatenate([x_cluster, beta], axis=1)
    return out, pred_energy


# speedup vs baseline: 1.1237x; 1.1237x over previous
import functools

import jax
import jax.numpy as jnp
from jax import lax
from jax.experimental import pallas as pl
from jax.experimental.pallas import tpu as pltpu
from jax.experimental.pallas import tpu_sc as plsc

_K_LIST = [16, 128, 16, 256]
_G = 128
_QPAD = 10240
_NW = 32
_NQW = _QPAD // _NW
_ROWBATCH = 16


def _elu(x):
    return jax.nn.elu(x)


def _bn(x, g, b, eps=1e-5):
    m = jnp.mean(x, axis=0)
    v = jnp.var(x, axis=0)
    return (x - m) / jnp.sqrt(v + eps) * g + b


def _thresh_body(k, st_ref, s_ref, t_ref, dt_ref):
    stb = st_ref[...]
    d0 = s_ref[:, 0:1] - stb[0, 0:1, :]
    d1 = s_ref[:, 1:2] - stb[0, 1:2, :]
    d2 = s_ref[:, 2:3] - stb[0, 2:3, :]
    dt = d0 * d0 + d1 * d1 + d2 * d2
    dt_ref[...] = dt
    di0 = lax.bitcast_convert_type(dt, jnp.int32)
    hi0 = jnp.max(di0, axis=0, keepdims=True)
    lo0 = jnp.zeros_like(hi0)

    def bis(_, c):
        lo, hi = c
        q = (hi - lo) >> 2
        m1 = lo + q
        m2 = lo + 2 * q
        m3 = lo + 3 * q
        di = lax.bitcast_convert_type(dt_ref[...], jnp.int32)
        c1 = jnp.sum(jnp.where(di <= m1, 1.0, 0.0), axis=0, keepdims=True)
        c2 = jnp.sum(jnp.where(di <= m2, 1.0, 0.0), axis=0, keepdims=True)
        c3 = jnp.sum(jnp.where(di <= m3, 1.0, 0.0), axis=0, keepdims=True)
        kf = float(k)
        g1 = c1 >= kf
        g2 = c2 >= kf
        g3 = c3 >= kf
        nlo = jnp.where(g1, lo, jnp.where(g2, m1 + 1, jnp.where(g3, m2 + 1, m3 + 1)))
        nhi = jnp.where(g1, m1, jnp.where(g2, m2, jnp.where(g3, m3, hi)))
        return nlo, nhi

    _, tb = lax.fori_loop(0, 17, bis, (lo0, hi0))
    t_ref[...] = lax.bitcast_convert_type(tb, jnp.float32).reshape(1, 1, _G)


@functools.partial(jax.jit, static_argnames=('k',))
def _thresholds(stg, s, k):
    n = s.shape[0]
    ngrid = _QPAD // _G
    t3 = pl.pallas_call(
        functools.partial(_thresh_body, k),
        grid=(ngrid,),
        in_specs=[
            pl.BlockSpec((1, 3, _G), lambda g: (g, 0, 0)),
            pl.BlockSpec((n, 3), lambda g: (0, 0)),
        ],
        out_specs=pl.BlockSpec((1, 1, _G), lambda g: (g, 0, 0)),
        out_shape=jax.ShapeDtypeStruct((ngrid, 1, _G), jnp.float32),
        scratch_shapes=[pltpu.VMEM((n, _G), jnp.float32)],
    )(stg, s)
    return t3.reshape(_QPAD)


def _lane_splat(vec, i):
    idx = jnp.full((16, 1), i, jnp.int32)
    return lax.gather(
        vec, idx,
        dimension_numbers=lax.GatherDimensionNumbers(
            offset_dims=(), collapsed_slice_dims=(0,), start_index_map=(0,)),
        slice_sizes=(1,), mode=lax.GatherScatterMode.PROMISE_IN_BOUNDS)


def _sc_body(C, n_real, s0h, s1h, s2h, th, d2h, jh,
             s0, s1, s2, tv, rowd, rowj):
    NC = 2
    wid = lax.axis_index("s") * NC + lax.axis_index("c")
    pltpu.sync_copy(s0h, s0)
    pltpu.sync_copy(s1h, s1)
    pltpu.sync_copy(s2h, s2)
    pltpu.sync_copy(th, tv)
    qbase = wid * _NQW
    lane = lax.iota(jnp.int32, 16)
    nchunk = n_real // 16
    sent_d = jnp.full((16,), 1e30, jnp.float32)
    sent_j = jnp.zeros((16,), jnp.int32)
    nbuf = (_ROWBATCH * C) // 16

    def qgroup(gi, _):
        def clr(bi, _):
            rowd[pl.ds(bi * 16, 16)] = sent_d
            rowj[pl.ds(bi * 16, 16)] = sent_j
            return 0
        lax.fori_loop(0, nbuf, clr, 0)

        qg = qbase + gi * _ROWBATCH
        qv0 = s0[pl.ds(qg, 16)]
        qv1 = s1[pl.ds(qg, 16)]
        qv2 = s2[pl.ds(qg, 16)]
        tvv = tv[pl.ds(qg, 16)]
        for qi in range(_ROWBATCH):
            q0 = _lane_splat(qv0, qi)
            q1 = _lane_splat(qv1, qi)
            q2 = _lane_splat(qv2, qi)
            tq = _lane_splat(tvv, qi)
            rbase = qi * C
            cap = jnp.full((16,), qi * C + C - 1, jnp.int32)

            def chunk(c, obase, q0=q0, q1=q1, q2=q2, tq=tq,
                      rbase=rbase, cap=cap):
                e0 = s0[pl.ds(c * 16, 16)] - q0
                e1 = s1[pl.ds(c * 16, 16)] - q1
                e2 = s2[pl.ds(c * 16, 16)] - q2
                dv = e0 * e0 + e1 * e1 + e2 * e2
                m = dv <= tq
                mi = jnp.where(m, 1, 0)
                cum = plsc.cumsum(mi)
                pos = jnp.minimum(obase + (cum - mi), cap)
                plsc.store_scatter(rowd, [pos], dv, mask=m)
                plsc.store_scatter(rowj, [pos], lane + c * 16, mask=m)
                pc = plsc.all_reduce_population_count(m)
                return jnp.minimum(obase + pc, cap)

            lax.fori_loop(0, nchunk, chunk,
                          jnp.full((16,), rbase, jnp.int32))

        rq = qbase + gi * _ROWBATCH
        pltpu.sync_copy(rowd, d2h.at[pl.ds(rq * C, _ROWBATCH * C)])
        pltpu.sync_copy(rowj, jh.at[pl.ds(rq * C, _ROWBATCH * C)])
        return 0

    lax.fori_loop(0, _NQW // _ROWBATCH, qgroup, 0)


@functools.partial(jax.jit, static_argnames=('C', 'n_real'))
def _sc_compact(s0, s1, s2, t, C, n_real):
    mesh = plsc.VectorSubcoreMesh(core_axis_name="c", subcore_axis_name="s")
    kfn = pl.kernel(
        functools.partial(_sc_body, C, n_real),
        out_type=[
            jax.ShapeDtypeStruct((_QPAD * C,), jnp.float32),
            jax.ShapeDtypeStruct((_QPAD * C,), jnp.int32),
        ],
        mesh=mesh,
        compiler_params=pltpu.CompilerParams(needs_layout_passes=False),
        scratch_types=[
            pltpu.VMEM((_QPAD,), jnp.float32),
            pltpu.VMEM((_QPAD,), jnp.float32),
            pltpu.VMEM((_QPAD,), jnp.float32),
            pltpu.VMEM((_QPAD,), jnp.float32),
            pltpu.VMEM((_ROWBATCH * C,), jnp.float32),
            pltpu.VMEM((_ROWBATCH * C,), jnp.int32),
        ],
    )
    return kfn(s0, s1, s2, t)


def _knn_pallas(s, k):
    n = s.shape[0]
    C = k + 16
    stp = jnp.zeros((3, _QPAD), s.dtype).at[:, :n].set(s.T)
    stg = stp.reshape(3, _QPAD // _G, _G).transpose(1, 0, 2)
    t = _thresholds(stg, s, k)
    d2f, jf = _sc_compact(stp[0], stp[1], stp[2], t, C, n)
    d2c = d2f.reshape(_QPAD, C)[:n]
    jc = jf.reshape(_QPAD, C)[:n]
    nd2, pos = lax.top_k(-d2c, k)
    idx = jnp.take_along_axis(jc, pos, axis=1)
    return -nd2, idx


def _gravnet_conv(x, p, k):
    s = x @ p['Ws'] + p['bs']
    h = x @ p['Wh'] + p['bh']
    d2, idx = _knn_pallas(s, k)
    w = jnp.exp(-10.0 * d2)[..., None]
    msg = h[idx] * w
    mean_agg = jnp.mean(msg, axis=1)
    max_agg = jnp.max(msg, axis=1)
    out = jnp.concatenate([x, mean_agg, max_agg], axis=1) @ p['Wo'] + p['bo']
    return out, s


def _block_fn(x_in, p, k):
    x = _elu(x_in @ p['pre1W'] + p['pre1b'])
    x = _elu(x @ p['pre2W'] + p['pre2b'])
    x = _bn(x, p['bn1g'], p['bn1b'])
    x_input = x
    xgn, s = _gravnet_conv(x, p, k)
    x = jnp.concatenate([xgn, s, x_input], axis=1)
    x = _elu(x @ p['post1W'] + p['post1b'])
    x = _elu(x @ p['post2W'] + p['post2b'])
    x = _bn(x, p['bn2g'], p['bn2b'])
    return x


def kernel(x, step_count, params):
    x = _bn(x, params['bn0_g'], params['bn0_b'])
    x = x @ params['Wd1']
    allfeat = [x]
    cur = x
    for i, k in enumerate(_K_LIST):
        xi = _block_fn(cur, params['block%d' % i], k)
        allfeat.append(xi)
        cur = jnp.concatenate(allfeat, axis=1)
    x = jnp.concatenate(allfeat, axis=-1)
    for j in range(3):
        x = _elu(x @ params['denseW%d' % j] + params['denseb%d' % j])
    x = _bn(x, params['bn2_g'], params['bn2_b'])
    x_cluster = x @ params['Wclust']
    pred_energy = 1.0 + _elu(x @ params['Wpe'])
    beta = x @ params['Wbeta'] + params['bbeta']
    out = jnp.concatenate([x_cluster, beta], axis=1)
    return out, pred_energy
